# baseline (speedup 1.0000x reference)
import jax
import jax.numpy as jnp
from jax.experimental import pallas as pl
from jax.experimental.pallas import tpu as pltpu

BT = 512
BN = 256


def _ffn_kernel(x_ref, w1_ref, b1_ref, w2_ref, b2_ref, o_ref, h_ref):
    n = pl.program_id(1)

    @pl.when(n == 0)
    def _():
        h = jnp.dot(x_ref[...], w1_ref[...], preferred_element_type=jnp.float32)
        h = h + b1_ref[...]
        h_ref[...] = h * 0.5 * (1.0 + jax.lax.erf(h * 0.7071067811865476))

    o_ref[...] = jnp.dot(h_ref[...], w2_ref[...],
                         preferred_element_type=jnp.float32) + b2_ref[...]


def kernel(inputs, W1, b1, W2, b2, input_split):
    del input_split
    tokens, d_model = inputs.shape
    d_ff = W1.shape[1]
    b1_2d = b1.reshape(1, d_ff)
    b2_2d = b2.reshape(1, d_model)
    return pl.pallas_call(
        _ffn_kernel,
        grid=(tokens // BT, d_model // BN),
        in_specs=[
            pl.BlockSpec((BT, d_model), lambda i, n: (i, 0)),
            pl.BlockSpec((d_model, d_ff), lambda i, n: (0, 0)),
            pl.BlockSpec((1, d_ff), lambda i, n: (0, 0)),
            pl.BlockSpec((d_ff, BN), lambda i, n: (0, n)),
            pl.BlockSpec((1, BN), lambda i, n: (0, n)),
        ],
        out_specs=pl.BlockSpec((BT, BN), lambda i, n: (i, n)),
        out_shape=jax.ShapeDtypeStruct((tokens, d_model), jnp.float32),
        scratch_shapes=[pltpu.VMEM((BT, d_ff), jnp.float32)],
        compiler_params=pltpu.CompilerParams(
            dimension_semantics=("parallel", "arbitrary"),
            vmem_limit_bytes=100 * 1024 * 1024),
    )(inputs, W1, b1_2d, W2, b2_2d)


# speedup vs baseline: 1.2078x; 1.2078x over previous
import jax
import jax.numpy as jnp
from jax.experimental import pallas as pl
from jax.experimental.pallas import tpu as pltpu

BT = 1024
BF = 512


def _ffn_kernel(x_ref, w1_ref, b1_ref, w2_ref, b2_ref, o_ref):
    j = pl.program_id(1)
    h = jnp.dot(x_ref[...], w1_ref[...], preferred_element_type=jnp.float32)
    h = h + b1_ref[...]
    h = h * 0.5 * (1.0 + jax.lax.erf(h * 0.7071067811865476))
    contrib = jnp.dot(h, w2_ref[...], preferred_element_type=jnp.float32)

    @pl.when(j == 0)
    def _():
        o_ref[...] = contrib + b2_ref[...]

    @pl.when(j != 0)
    def _():
        o_ref[...] = o_ref[...] + contrib


def kernel(inputs, W1, b1, W2, b2, input_split):
    del input_split
    tokens, d_model = inputs.shape
    d_ff = W1.shape[1]
    b1_2d = b1.reshape(1, d_ff)
    b2_2d = b2.reshape(1, d_model)
    return pl.pallas_call(
        _ffn_kernel,
        grid=(tokens // BT, d_ff // BF),
        in_specs=[
            pl.BlockSpec((BT, d_model), lambda i, j: (i, 0)),
            pl.BlockSpec((d_model, BF), lambda i, j: (0, j)),
            pl.BlockSpec((1, BF), lambda i, j: (0, j)),
            pl.BlockSpec((BF, d_model), lambda i, j: (j, 0)),
            pl.BlockSpec((1, d_model), lambda i, j: (0, 0)),
        ],
        out_specs=pl.BlockSpec((BT, d_model), lambda i, j: (i, 0)),
        out_shape=jax.ShapeDtypeStruct((tokens, d_model), jnp.float32),
        compiler_params=pltpu.CompilerParams(
            dimension_semantics=("arbitrary", "arbitrary"),
            vmem_limit_bytes=100 * 1024 * 1024),
    )(inputs, W1, b1_2d, W2, b2_2d)
